# initial kernel scaffold (unmeasured)
import jax
import jax.numpy as jnp
from jax import lax
from jax.experimental import pallas as pl
from jax.experimental.pallas import tpu as pltpu

N_DEV = 16
B, S, D = 2, 256, 512
NH_LOCAL = 4
DH = 64
ROWS = B * S
CHUNK = ROWS // N_DEV
EPS = 1e-5
F32 = jnp.float32


def kernel(x, Wq, Wk, Wv, Wo, t_emb, W_mod, W_ff1, W_ff2):
    def body(x_ref, wq_ref, wk_ref, wv_ref, wo_ref, temb_ref, wmod_ref,
             wff1_ref, wff2_ref, out_ref,
             acc_ref, rbuf_ref, res_ref,
             rs_send, rs_recv, ag_send, ag_recv):
        my = lax.axis_index("i")
        left = lax.rem(my + N_DEV - 1, N_DEV)
        right = lax.rem(my + 1, N_DEV)

        barrier = pltpu.get_barrier_semaphore()
        for nbr in (left, right):
            pl.semaphore_signal(
                barrier, inc=1,
                device_id=(nbr,), device_id_type=pl.DeviceIdType.MESH,
            )
        pl.semaphore_wait(barrier, 2)

        def layer_norm(h):
            mu = jnp.mean(h, axis=-1, keepdims=True)
            var = jnp.mean((h - mu) * (h - mu), axis=-1, keepdims=True)
            return (h - mu) * lax.rsqrt(var + EPS)

        def ring_allreduce(phase):
            for h in range(N_DEV - 1):
                s_idx = lax.rem(my - h + 2 * N_DEV, N_DEV)
                rdma = pltpu.make_async_remote_copy(
                    src_ref=acc_ref.at[pl.ds(s_idx * CHUNK, CHUNK), :],
                    dst_ref=rbuf_ref.at[h],
                    send_sem=rs_send.at[phase, h],
                    recv_sem=rs_recv.at[phase, h],
                    device_id=(right,),
                    device_id_type=pl.DeviceIdType.MESH,
                )
                rdma.start()
                rdma.wait()
                r_idx = lax.rem(my - 1 - h + 2 * N_DEV, N_DEV)
                cur = pl.load(acc_ref, (pl.ds(r_idx * CHUNK, CHUNK), slice(None)))
                pl.store(acc_ref, (pl.ds(r_idx * CHUNK, CHUNK), slice(None)),
                         cur + rbuf_ref[h])

            own = lax.rem(my + 1, N_DEV)
            pl.store(res_ref, (pl.ds(own * CHUNK, CHUNK), slice(None)),
                     pl.load(acc_ref, (pl.ds(own * CHUNK, CHUNK), slice(None))))

            for h in range(N_DEV - 1):
                g = lax.rem(my + 1 - h + 2 * N_DEV, N_DEV)
                rdma = pltpu.make_async_remote_copy(
                    src_ref=res_ref.at[pl.ds(g * CHUNK, CHUNK), :],
                    dst_ref=res_ref.at[pl.ds(g * CHUNK, CHUNK), :],
                    send_sem=ag_send.at[phase, h],
                    recv_sem=ag_recv.at[phase, h],
                    device_id=(right,),
                    device_id_type=pl.DeviceIdType.MESH,
                )
                rdma.start()
                rdma.wait()

        mod = jnp.dot(temb_ref[:], wmod_ref[:], preferred_element_type=F32)

        for b in range(B):
            sa = mod[b:b + 1, 0 * D:1 * D]
            sha = mod[b:b + 1, 1 * D:2 * D]
            xa = layer_norm(x_ref[b]) * (1.0 + sa) + sha
            q = jnp.dot(xa, wq_ref[:], preferred_element_type=F32)
            k = jnp.dot(xa, wk_ref[:], preferred_element_type=F32)
            v = jnp.dot(xa, wv_ref[:], preferred_element_type=F32)
            partial = jnp.zeros((S, D), dtype=F32)
            for hh in range(NH_LOCAL):
                lo, hi = hh * DH, (hh + 1) * DH
                s = lax.dot_general(
                    q[:, lo:hi], k[:, lo:hi],
                    (((1,), (1,)), ((), ())),
                    preferred_element_type=F32,
                ) * 0.125
                mx = jnp.max(s, axis=-1, keepdims=True)
                p = jnp.exp(s - mx)
                p = p / jnp.sum(p, axis=-1, keepdims=True)
                oh = jnp.dot(p, v[:, lo:hi], preferred_element_type=F32)
                partial = partial + jnp.dot(
                    oh, wo_ref[lo:hi, :], preferred_element_type=F32)
            acc_ref[b * S:(b + 1) * S, :] = partial

        ring_allreduce(0)

        for b in range(B):
            ga = mod[b:b + 1, 2 * D:3 * D]
            out_ref[b] = x_ref[b] + ga * res_ref[b * S:(b + 1) * S, :]

        for b in range(B):
            sm = mod[b:b + 1, 3 * D:4 * D]
            shm = mod[b:b + 1, 4 * D:5 * D]
            xc = layer_norm(out_ref[b]) * (1.0 + sm) + shm
            h1 = jnp.dot(xc, wff1_ref[:], preferred_element_type=F32)
            h1 = h1 * (1.0 / (1.0 + jnp.exp(-h1)))
            acc_ref[b * S:(b + 1) * S, :] = jnp.dot(
                h1, wff2_ref[:], preferred_element_type=F32)

        ring_allreduce(1)

        for b in range(B):
            gm = mod[b:b + 1, 5 * D:6 * D]
            out_ref[b] = out_ref[b] + gm * res_ref[b * S:(b + 1) * S, :]

    return pl.pallas_call(
        body,
        out_shape=jax.ShapeDtypeStruct((B, S, D), F32),
        in_specs=[pl.BlockSpec(memory_space=pltpu.VMEM)] * 9,
        out_specs=pl.BlockSpec(memory_space=pltpu.VMEM),
        scratch_shapes=[
            pltpu.VMEM((ROWS, D), F32),
            pltpu.VMEM((N_DEV - 1, CHUNK, D), F32),
            pltpu.VMEM((ROWS, D), F32),
            pltpu.SemaphoreType.DMA((2, N_DEV - 1)),
            pltpu.SemaphoreType.DMA((2, N_DEV - 1)),
            pltpu.SemaphoreType.DMA((2, N_DEV - 1)),
            pltpu.SemaphoreType.DMA((2, N_DEV - 1)),
        ],
        compiler_params=pltpu.CompilerParams(collective_id=0),
    )(x, Wq, Wk, Wv, Wo, t_emb, W_mod, W_ff1, W_ff2)


# baseline (device time: 172799 ns/iter reference)
import jax
import jax.numpy as jnp
from jax import lax
from jax.experimental import pallas as pl
from jax.experimental.pallas import tpu as pltpu

N_DEV = 16
B, S, D = 2, 256, 512
NH_LOCAL = 4
DH = 64
ROWS = B * S
CHUNK = ROWS // N_DEV
EPS = 1e-5
F32 = jnp.float32


def kernel(x, Wq, Wk, Wv, Wo, t_emb, W_mod, W_ff1, W_ff2):
    def body(x_ref, wq_ref, wk_ref, wv_ref, wo_ref, temb_ref, wmod_ref,
             wff1_ref, wff2_ref, out_ref,
             acc_ref, rbuf_ref, res_ref,
             rs_send, rs_recv, ag_send, ag_recv):
        my = lax.axis_index("i")
        left = lax.rem(my + N_DEV - 1, N_DEV)
        right = lax.rem(my + 1, N_DEV)

        barrier = pltpu.get_barrier_semaphore()
        for nbr in (left, right):
            pl.semaphore_signal(
                barrier, inc=1,
                device_id=(nbr,), device_id_type=pl.DeviceIdType.MESH,
            )
        pl.semaphore_wait(barrier, 2)

        def layer_norm(h):
            mu = jnp.mean(h, axis=-1, keepdims=True)
            var = jnp.mean((h - mu) * (h - mu), axis=-1, keepdims=True)
            return (h - mu) * lax.rsqrt(var + EPS)

        def ring_allreduce(phase):
            for h in range(N_DEV - 1):
                s_idx = lax.rem(my - h + 2 * N_DEV, N_DEV)
                rdma = pltpu.make_async_remote_copy(
                    src_ref=acc_ref.at[pl.ds(s_idx * CHUNK, CHUNK), :],
                    dst_ref=rbuf_ref.at[h],
                    send_sem=rs_send.at[phase, h],
                    recv_sem=rs_recv.at[phase, h],
                    device_id=(right,),
                    device_id_type=pl.DeviceIdType.MESH,
                )
                rdma.start()
                rdma.wait()
                r_idx = lax.rem(my - 1 - h + 2 * N_DEV, N_DEV)
                sl = pl.ds(r_idx * CHUNK, CHUNK)
                acc_ref[sl, :] = acc_ref[sl, :] + rbuf_ref[h]

            own_sl = pl.ds(lax.rem(my + 1, N_DEV) * CHUNK, CHUNK)
            res_ref[own_sl, :] = acc_ref[own_sl, :]

            for h in range(N_DEV - 1):
                g = lax.rem(my + 1 - h + 2 * N_DEV, N_DEV)
                rdma = pltpu.make_async_remote_copy(
                    src_ref=res_ref.at[pl.ds(g * CHUNK, CHUNK), :],
                    dst_ref=res_ref.at[pl.ds(g * CHUNK, CHUNK), :],
                    send_sem=ag_send.at[phase, h],
                    recv_sem=ag_recv.at[phase, h],
                    device_id=(right,),
                    device_id_type=pl.DeviceIdType.MESH,
                )
                rdma.start()
                rdma.wait()

        mod = jnp.dot(temb_ref[:], wmod_ref[:], preferred_element_type=F32)

        for b in range(B):
            sa = mod[b:b + 1, 0 * D:1 * D]
            sha = mod[b:b + 1, 1 * D:2 * D]
            xa = layer_norm(x_ref[b]) * (1.0 + sa) + sha
            q = jnp.dot(xa, wq_ref[:], preferred_element_type=F32)
            k = jnp.dot(xa, wk_ref[:], preferred_element_type=F32)
            v = jnp.dot(xa, wv_ref[:], preferred_element_type=F32)
            partial = jnp.zeros((S, D), dtype=F32)
            for hh in range(NH_LOCAL):
                lo, hi = hh * DH, (hh + 1) * DH
                s = lax.dot_general(
                    q[:, lo:hi], k[:, lo:hi],
                    (((1,), (1,)), ((), ())),
                    preferred_element_type=F32,
                ) * 0.125
                mx = jnp.max(s, axis=-1, keepdims=True)
                p = jnp.exp(s - mx)
                p = p / jnp.sum(p, axis=-1, keepdims=True)
                oh = jnp.dot(p, v[:, lo:hi], preferred_element_type=F32)
                partial = partial + jnp.dot(
                    oh, wo_ref[lo:hi, :], preferred_element_type=F32)
            acc_ref[b * S:(b + 1) * S, :] = partial

        ring_allreduce(0)

        for b in range(B):
            ga = mod[b:b + 1, 2 * D:3 * D]
            out_ref[b] = x_ref[b] + ga * res_ref[b * S:(b + 1) * S, :]

        for b in range(B):
            sm = mod[b:b + 1, 3 * D:4 * D]
            shm = mod[b:b + 1, 4 * D:5 * D]
            xc = layer_norm(out_ref[b]) * (1.0 + sm) + shm
            h1 = jnp.dot(xc, wff1_ref[:], preferred_element_type=F32)
            h1 = h1 * (1.0 / (1.0 + jnp.exp(-h1)))
            acc_ref[b * S:(b + 1) * S, :] = jnp.dot(
                h1, wff2_ref[:], preferred_element_type=F32)

        ring_allreduce(1)

        for b in range(B):
            gm = mod[b:b + 1, 5 * D:6 * D]
            out_ref[b] = out_ref[b] + gm * res_ref[b * S:(b + 1) * S, :]

    return pl.pallas_call(
        body,
        out_shape=jax.ShapeDtypeStruct((B, S, D), F32),
        in_specs=[pl.BlockSpec(memory_space=pltpu.VMEM)] * 9,
        out_specs=pl.BlockSpec(memory_space=pltpu.VMEM),
        scratch_shapes=[
            pltpu.VMEM((ROWS, D), F32),
            pltpu.VMEM((N_DEV - 1, CHUNK, D), F32),
            pltpu.VMEM((ROWS, D), F32),
            pltpu.SemaphoreType.DMA((2, N_DEV - 1)),
            pltpu.SemaphoreType.DMA((2, N_DEV - 1)),
            pltpu.SemaphoreType.DMA((2, N_DEV - 1)),
            pltpu.SemaphoreType.DMA((2, N_DEV - 1)),
        ],
        compiler_params=pltpu.CompilerParams(collective_id=0),
    )(x, Wq, Wk, Wv, Wo, t_emb, W_mod, W_ff1, W_ff2)


# device time: 70569 ns/iter; 2.4487x vs baseline; 2.4487x over previous
import jax
import jax.numpy as jnp
from jax import lax
from jax.experimental import pallas as pl
from jax.experimental.pallas import tpu as pltpu

N_DEV = 16
B, S, D = 2, 256, 512
NH_LOCAL = 4
DH = 64
ROWS = B * S
CHUNK = ROWS // N_DEV
EPS = 1e-5
F32 = jnp.float32


def kernel(x, Wq, Wk, Wv, Wo, t_emb, W_mod, W_ff1, W_ff2):
    def body(x_ref, wq_ref, wk_ref, wv_ref, wo_ref, temb_ref, wmod_ref,
             wff1_ref, wff2_ref, out_ref,
             acc_ref, rbuf_ref, res_ref,
             rs_send, rs_recv, ag_send, ag_recv):
        my = lax.axis_index("i")
        left = lax.rem(my + N_DEV - 1, N_DEV)
        right = lax.rem(my + 1, N_DEV)

        barrier = pltpu.get_barrier_semaphore()
        for o in range(1, N_DEV):
            pl.semaphore_signal(
                barrier, inc=1,
                device_id=(lax.rem(my + o, N_DEV),),
                device_id_type=pl.DeviceIdType.MESH,
            )
        pl.semaphore_wait(barrier, N_DEV - 1)

        def layer_norm(h):
            mu = jnp.mean(h, axis=-1, keepdims=True)
            var = jnp.mean((h - mu) * (h - mu), axis=-1, keepdims=True)
            return (h - mu) * lax.rsqrt(var + EPS)

        def ring_allreduce(phase):
            my_sl = pl.ds(my * CHUNK, CHUNK)

            rdmas = []
            for o in range(1, N_DEV):
                d = lax.rem(my + o, N_DEV)
                rdma = pltpu.make_async_remote_copy(
                    src_ref=acc_ref.at[pl.ds(d * CHUNK, CHUNK), :],
                    dst_ref=rbuf_ref.at[N_DEV - 1 - o],
                    send_sem=rs_send.at[phase, o - 1],
                    recv_sem=rs_recv.at[phase, N_DEV - 1 - o],
                    device_id=(d,),
                    device_id_type=pl.DeviceIdType.MESH,
                )
                rdma.start()
                rdmas.append(rdma)
            for rdma in rdmas:
                rdma.wait()

            red = acc_ref[my_sl, :]
            for k in range(N_DEV - 1):
                red = red + rbuf_ref[k]
            res_ref[my_sl, :] = red

            rdmas = []
            for o in range(1, N_DEV):
                d = lax.rem(my + o, N_DEV)
                rdma = pltpu.make_async_remote_copy(
                    src_ref=res_ref.at[my_sl, :],
                    dst_ref=res_ref.at[my_sl, :],
                    send_sem=ag_send.at[phase, o - 1],
                    recv_sem=ag_recv.at[phase, N_DEV - 1 - o],
                    device_id=(d,),
                    device_id_type=pl.DeviceIdType.MESH,
                )
                rdma.start()
                rdmas.append(rdma)
            for rdma in rdmas:
                rdma.wait()

        mod = jnp.dot(temb_ref[:], wmod_ref[:], preferred_element_type=F32)

        for b in range(B):
            sa = mod[b:b + 1, 0 * D:1 * D]
            sha = mod[b:b + 1, 1 * D:2 * D]
            xa = layer_norm(x_ref[b]) * (1.0 + sa) + sha
            q = jnp.dot(xa, wq_ref[:], preferred_element_type=F32)
            k = jnp.dot(xa, wk_ref[:], preferred_element_type=F32)
            v = jnp.dot(xa, wv_ref[:], preferred_element_type=F32)
            partial = jnp.zeros((S, D), dtype=F32)
            for hh in range(NH_LOCAL):
                lo, hi = hh * DH, (hh + 1) * DH
                s = lax.dot_general(
                    q[:, lo:hi], k[:, lo:hi],
                    (((1,), (1,)), ((), ())),
                    preferred_element_type=F32,
                ) * 0.125
                mx = jnp.max(s, axis=-1, keepdims=True)
                p = jnp.exp(s - mx)
                p = p / jnp.sum(p, axis=-1, keepdims=True)
                oh = jnp.dot(p, v[:, lo:hi], preferred_element_type=F32)
                partial = partial + jnp.dot(
                    oh, wo_ref[lo:hi, :], preferred_element_type=F32)
            acc_ref[b * S:(b + 1) * S, :] = partial

        ring_allreduce(0)

        for b in range(B):
            ga = mod[b:b + 1, 2 * D:3 * D]
            out_ref[b] = x_ref[b] + ga * res_ref[b * S:(b + 1) * S, :]

        for b in range(B):
            sm = mod[b:b + 1, 3 * D:4 * D]
            shm = mod[b:b + 1, 4 * D:5 * D]
            xc = layer_norm(out_ref[b]) * (1.0 + sm) + shm
            h1 = jnp.dot(xc, wff1_ref[:], preferred_element_type=F32)
            h1 = h1 * (1.0 / (1.0 + jnp.exp(-h1)))
            acc_ref[b * S:(b + 1) * S, :] = jnp.dot(
                h1, wff2_ref[:], preferred_element_type=F32)

        ring_allreduce(1)

        for b in range(B):
            gm = mod[b:b + 1, 5 * D:6 * D]
            out_ref[b] = out_ref[b] + gm * res_ref[b * S:(b + 1) * S, :]

    return pl.pallas_call(
        body,
        out_shape=jax.ShapeDtypeStruct((B, S, D), F32),
        in_specs=[pl.BlockSpec(memory_space=pltpu.VMEM)] * 9,
        out_specs=pl.BlockSpec(memory_space=pltpu.VMEM),
        scratch_shapes=[
            pltpu.VMEM((ROWS, D), F32),
            pltpu.VMEM((N_DEV - 1, CHUNK, D), F32),
            pltpu.VMEM((ROWS, D), F32),
            pltpu.SemaphoreType.DMA((2, N_DEV - 1)),
            pltpu.SemaphoreType.DMA((2, N_DEV - 1)),
            pltpu.SemaphoreType.DMA((2, N_DEV - 1)),
            pltpu.SemaphoreType.DMA((2, N_DEV - 1)),
        ],
        compiler_params=pltpu.CompilerParams(collective_id=0),
    )(x, Wq, Wk, Wv, Wo, t_emb, W_mod, W_ff1, W_ff2)


# device time: 51470 ns/iter; 3.3573x vs baseline; 1.3711x over previous
import jax
import jax.numpy as jnp
from jax import lax
from jax.experimental import pallas as pl
from jax.experimental.pallas import tpu as pltpu

N_DEV = 16
B, S, D = 2, 256, 512
NH_LOCAL = 4
DH = 64
ROWS = B * S
CHUNK = ROWS // N_DEV
EPS = 1e-5
F32 = jnp.float32


def kernel(x, Wq, Wk, Wv, Wo, t_emb, W_mod, W_ff1, W_ff2):
    def body(x_ref, wq_ref, wk_ref, wv_ref, wo_ref, temb_ref, wmod_ref,
             wff1_ref, wff2_ref, out_ref,
             sbuf_ref, rbuf_ref, res_ref,
             rs_send, rs_recv, ag_send, ag_recv):
        my = lax.axis_index("i")
        left = lax.rem(my + N_DEV - 1, N_DEV)
        right = lax.rem(my + 1, N_DEV)

        barrier = pltpu.get_barrier_semaphore()
        for o in range(1, N_DEV):
            pl.semaphore_signal(
                barrier, inc=1,
                device_id=(lax.rem(my + o, N_DEV),),
                device_id_type=pl.DeviceIdType.MESH,
            )
        pl.semaphore_wait(barrier, N_DEV - 1)

        def layer_norm(h):
            mu = jnp.mean(h, axis=-1, keepdims=True)
            var = jnp.mean((h - mu) * (h - mu), axis=-1, keepdims=True)
            return (h - mu) * lax.rsqrt(var + EPS)

        def ring_allreduce(phase):
            my_sl = pl.ds(my * CHUNK, CHUNK)

            rdmas = []
            for o in range(1, N_DEV):
                d = lax.rem(my + o, N_DEV)
                rdma = pltpu.make_async_remote_copy(
                    src_ref=sbuf_ref.at[pl.ds(d * CHUNK, CHUNK), :],
                    dst_ref=rbuf_ref.at[N_DEV - 1 - o],
                    send_sem=rs_send.at[phase, o - 1],
                    recv_sem=rs_recv.at[phase, N_DEV - 1 - o],
                    device_id=(d,),
                    device_id_type=pl.DeviceIdType.MESH,
                )
                rdma.start()
                rdmas.append(rdma)
            for rdma in rdmas:
                rdma.wait()

            red = sbuf_ref[my_sl, :].astype(F32)
            for k in range(N_DEV - 1):
                red = red + rbuf_ref[k].astype(F32)
            res_ref[my_sl, :] = red.astype(jnp.bfloat16)

            rdmas = []
            for o in range(1, N_DEV):
                d = lax.rem(my + o, N_DEV)
                rdma = pltpu.make_async_remote_copy(
                    src_ref=res_ref.at[my_sl, :],
                    dst_ref=res_ref.at[my_sl, :],
                    send_sem=ag_send.at[phase, o - 1],
                    recv_sem=ag_recv.at[phase, N_DEV - 1 - o],
                    device_id=(d,),
                    device_id_type=pl.DeviceIdType.MESH,
                )
                rdma.start()
                rdmas.append(rdma)
            for rdma in rdmas:
                rdma.wait()

        mod = jnp.dot(temb_ref[:], wmod_ref[:], preferred_element_type=F32)

        for b in range(B):
            sa = mod[b:b + 1, 0 * D:1 * D]
            sha = mod[b:b + 1, 1 * D:2 * D]
            xa = layer_norm(x_ref[b]) * (1.0 + sa) + sha
            q = jnp.dot(xa, wq_ref[:], preferred_element_type=F32)
            k = jnp.dot(xa, wk_ref[:], preferred_element_type=F32)
            v = jnp.dot(xa, wv_ref[:], preferred_element_type=F32)
            partial = jnp.zeros((S, D), dtype=F32)
            for hh in range(NH_LOCAL):
                lo, hi = hh * DH, (hh + 1) * DH
                s = lax.dot_general(
                    q[:, lo:hi], k[:, lo:hi],
                    (((1,), (1,)), ((), ())),
                    preferred_element_type=F32,
                ) * 0.125
                mx = jnp.max(s, axis=-1, keepdims=True)
                p = jnp.exp(s - mx)
                p = p / jnp.sum(p, axis=-1, keepdims=True)
                oh = jnp.dot(p, v[:, lo:hi], preferred_element_type=F32)
                partial = partial + jnp.dot(
                    oh, wo_ref[lo:hi, :], preferred_element_type=F32)
            sbuf_ref[b * S:(b + 1) * S, :] = partial.astype(jnp.bfloat16)

        ring_allreduce(0)

        for b in range(B):
            ga = mod[b:b + 1, 2 * D:3 * D]
            out_ref[b] = x_ref[b] + ga * res_ref[b * S:(b + 1) * S, :].astype(F32)

        for b in range(B):
            sm = mod[b:b + 1, 3 * D:4 * D]
            shm = mod[b:b + 1, 4 * D:5 * D]
            xc = layer_norm(out_ref[b]) * (1.0 + sm) + shm
            h1 = jnp.dot(xc, wff1_ref[:], preferred_element_type=F32)
            h1 = h1 * (1.0 / (1.0 + jnp.exp(-h1)))
            sbuf_ref[b * S:(b + 1) * S, :] = jnp.dot(
                h1, wff2_ref[:], preferred_element_type=F32).astype(jnp.bfloat16)

        ring_allreduce(1)

        for b in range(B):
            gm = mod[b:b + 1, 5 * D:6 * D]
            out_ref[b] = out_ref[b] + gm * res_ref[b * S:(b + 1) * S, :].astype(F32)

    return pl.pallas_call(
        body,
        out_shape=jax.ShapeDtypeStruct((B, S, D), F32),
        in_specs=[pl.BlockSpec(memory_space=pltpu.VMEM)] * 9,
        out_specs=pl.BlockSpec(memory_space=pltpu.VMEM),
        scratch_shapes=[
            pltpu.VMEM((ROWS, D), jnp.bfloat16),
            pltpu.VMEM((N_DEV - 1, CHUNK, D), jnp.bfloat16),
            pltpu.VMEM((ROWS, D), jnp.bfloat16),
            pltpu.SemaphoreType.DMA((2, N_DEV - 1)),
            pltpu.SemaphoreType.DMA((2, N_DEV - 1)),
            pltpu.SemaphoreType.DMA((2, N_DEV - 1)),
            pltpu.SemaphoreType.DMA((2, N_DEV - 1)),
        ],
        compiler_params=pltpu.CompilerParams(collective_id=0),
    )(x, Wq, Wk, Wv, Wo, t_emb, W_mod, W_ff1, W_ff2)


# device time: 51039 ns/iter; 3.3856x vs baseline; 1.0084x over previous
import jax
import jax.numpy as jnp
from jax import lax
from jax.experimental import pallas as pl
from jax.experimental.pallas import tpu as pltpu

N_DEV = 16
_PROBE_NO_ATTN = False
_PROBE_MIN = False
_PROBE_NO_BARRIER = False
B, S, D = 2, 256, 512
NH_LOCAL = 4
DH = 64
ROWS = B * S
CHUNK = ROWS // N_DEV
CPB = S // CHUNK
EPS = 1e-5
F32 = jnp.float32
BF16 = jnp.bfloat16


def kernel(x, Wq, Wk, Wv, Wo, t_emb, W_mod, W_ff1, W_ff2):
    def body(x_ref, wq_ref, wk_ref, wv_ref, wo_ref, temb_ref, wmod_ref,
             wff1_ref, wff2_ref, out_ref,
             sbuf0_ref, sbuf1_ref, rbuf_ref, res_ref,
             rs_send, rs_recv, ag_send, ag_recv):
        sbufs = (sbuf0_ref, sbuf1_ref)
        my = lax.axis_index("i")

        if not _PROBE_NO_BARRIER:
            barrier = pltpu.get_barrier_semaphore()
            for o in range(1, N_DEV):
                pl.semaphore_signal(
                    barrier, inc=1,
                    device_id=(lax.rem(my + o, N_DEV),),
                    device_id_type=pl.DeviceIdType.MESH,
                )
            pl.semaphore_wait(barrier, N_DEV - 1)

        if _PROBE_MIN:
            out_ref[0] = x_ref[0] * wq_ref[0, 0] * wk_ref[0, 0] * wv_ref[0, 0] \
                * wo_ref[0, 0] * temb_ref[0, 0] * wmod_ref[0, 0] \
                * wff1_ref[0, 0] * wff2_ref[0, 0]
            out_ref[1] = x_ref[1]
            return

        def layer_norm(h):
            mu = jnp.mean(h, axis=-1, keepdims=True)
            var = jnp.mean((h - mu) * (h - mu), axis=-1, keepdims=True)
            return (h - mu) * lax.rsqrt(var + EPS)

        def issue_rs(phase, c0, c1):
            rdmas = []
            for c in range(c0, c1):
                slot = lax.rem(my - c + N_DEV, N_DEV)
                rdma = pltpu.make_async_remote_copy(
                    src_ref=sbufs[c // CPB].at[pl.ds((c % CPB) * CHUNK, CHUNK), :],
                    dst_ref=rbuf_ref.at[slot],
                    send_sem=rs_send.at[phase, c],
                    recv_sem=rs_recv.at[phase, slot],
                    device_id=(c,),
                    device_id_type=pl.DeviceIdType.MESH,
                )
                rdma.start()
                rdmas.append(rdma)
            return rdmas

        def finish_allreduce(phase, rdmas):
            for rdma in rdmas:
                rdma.wait()

            my_sl = pl.ds(my * CHUNK, CHUNK)
            red = rbuf_ref[0].astype(F32)
            for k in range(1, N_DEV):
                red = red + rbuf_ref[k].astype(F32)
            res_ref[my_sl, :] = red.astype(BF16)

            rdmas = []
            for o in range(1, N_DEV):
                d = lax.rem(my + o, N_DEV)
                rdma = pltpu.make_async_remote_copy(
                    src_ref=res_ref.at[my_sl, :],
                    dst_ref=res_ref.at[my_sl, :],
                    send_sem=ag_send.at[phase, o - 1],
                    recv_sem=ag_recv.at[phase, N_DEV - 1 - o],
                    device_id=(d,),
                    device_id_type=pl.DeviceIdType.MESH,
                )
                rdma.start()
                rdmas.append(rdma)
            for rdma in rdmas:
                rdma.wait()

        mod = jnp.dot(temb_ref[:], wmod_ref[:], preferred_element_type=F32)

        rs1 = []
        for b in range(B):
            sa = mod[b:b + 1, 0 * D:1 * D]
            sha = mod[b:b + 1, 1 * D:2 * D]
            xa = layer_norm(x_ref[b]) * (1.0 + sa) + sha
            q = jnp.dot(xa, wq_ref[:], preferred_element_type=F32)
            k = jnp.dot(xa, wk_ref[:], preferred_element_type=F32)
            v = jnp.dot(xa, wv_ref[:], preferred_element_type=F32)
            if _PROBE_NO_ATTN:
                partial = jnp.dot(q + v, wo_ref[:], preferred_element_type=F32)
                sbufs[b][:] = partial.astype(BF16)
                rs1 += issue_rs(0, b * CPB, (b + 1) * CPB)
                continue
            partial = jnp.zeros((S, D), dtype=F32)
            for hh in range(NH_LOCAL):
                lo, hi = hh * DH, (hh + 1) * DH
                s = lax.dot_general(
                    q[:, lo:hi], k[:, lo:hi],
                    (((1,), (1,)), ((), ())),
                    preferred_element_type=F32,
                ) * 0.125
                mx = jnp.max(s, axis=-1, keepdims=True)
                p = jnp.exp(s - mx)
                p = p / jnp.sum(p, axis=-1, keepdims=True)
                oh = jnp.dot(p, v[:, lo:hi], preferred_element_type=F32)
                partial = partial + jnp.dot(
                    oh, wo_ref[lo:hi, :], preferred_element_type=F32)
            sbufs[b][:] = partial.astype(BF16)
            rs1 += issue_rs(0, b * CPB, (b + 1) * CPB)

        finish_allreduce(0, rs1)

        for b in range(B):
            ga = mod[b:b + 1, 2 * D:3 * D]
            out_ref[b] = x_ref[b] + ga * res_ref[b * S:(b + 1) * S, :].astype(F32)

        rs2 = []
        for b in range(B):
            sm = mod[b:b + 1, 3 * D:4 * D]
            shm = mod[b:b + 1, 4 * D:5 * D]
            xc = layer_norm(out_ref[b]) * (1.0 + sm) + shm
            h1 = jnp.dot(xc, wff1_ref[:], preferred_element_type=F32)
            h1 = h1 * (1.0 / (1.0 + jnp.exp(-h1)))
            sbufs[b][:] = jnp.dot(
                h1, wff2_ref[:], preferred_element_type=F32).astype(BF16)
            rs2 += issue_rs(1, b * CPB, (b + 1) * CPB)

        finish_allreduce(1, rs2)

        for b in range(B):
            gm = mod[b:b + 1, 5 * D:6 * D]
            out_ref[b] = out_ref[b] + gm * res_ref[b * S:(b + 1) * S, :].astype(F32)

    return pl.pallas_call(
        body,
        out_shape=jax.ShapeDtypeStruct((B, S, D), F32),
        in_specs=[pl.BlockSpec(memory_space=pltpu.VMEM)] * 9,
        out_specs=pl.BlockSpec(memory_space=pltpu.VMEM),
        scratch_shapes=[
            pltpu.VMEM((S, D), BF16),
            pltpu.VMEM((S, D), BF16),
            pltpu.VMEM((N_DEV, CHUNK, D), BF16),
            pltpu.VMEM((ROWS, D), BF16),
            pltpu.SemaphoreType.DMA((2, N_DEV)),
            pltpu.SemaphoreType.DMA((2, N_DEV)),
            pltpu.SemaphoreType.DMA((2, N_DEV - 1)),
            pltpu.SemaphoreType.DMA((2, N_DEV - 1)),
        ],
        compiler_params=pltpu.CompilerParams(
            collective_id=None if _PROBE_NO_BARRIER else 0),
    )(x, Wq, Wk, Wv, Wo, t_emb, W_mod, W_ff1, W_ff2)
